# baseline (speedup 1.0000x reference)
import functools

import jax
import jax.numpy as jnp
from jax.experimental import pallas as pl
from jax.experimental.pallas import tpu as pltpu

F32 = jnp.float32
GTU_KS = (3, 5, 7)

B = 8
N = 8
BN = B * N
CIN = 2
C = 32
SC = 64
SEQ = 12
T0 = 19
NLAYERS = 3
K = 3
OUT_DIM = 12
EPS = 1e-5

TS = (19, 13, 7)

_MROWS = []
for _i in range(NLAYERS):
    _t = TS[_i]
    for _k in GTU_KS:
        _MROWS.append((_t, _t - _k + 1))
    for _k in GTU_KS:
        _MROWS.append((_t - 6, _t - _k + 1))


def _dot(a, b):
    return jnp.dot(a, b, preferred_element_type=F32)


def _dot_bt(a, b):
    return jax.lax.dot_general(a, b, (((1,), (1,)), ((), ())),
                               preferred_element_type=F32)


def _dot_tb(a, b):
    return jax.lax.dot_general(a, b, (((0,), (0,)), ((), ())),
                               preferred_element_type=F32)


def _softmax0(x):
    m = jnp.max(x, axis=0, keepdims=True)
    e = jnp.exp(x - m)
    return e / jnp.sum(e, axis=0, keepdims=True)


def _iota(shape, d):
    return jax.lax.broadcasted_iota(jnp.int32, shape, d)


def _eqf(a, b):
    return (a == b).astype(F32)


def _gtu_bank(X, T, wpq, bp, bq):
    outs = []
    tap = 0
    for j, k in enumerate(GTU_KS):
        rows = (T - k + 1) * BN
        acc = jnp.zeros((rows, 2 * C), F32)
        for dt in range(k):
            acc = acc + _dot(X[dt * BN: dt * BN + rows, :],
                             wpq[(tap + dt) * C:(tap + dt + 1) * C, :])
        tap += k
        p = acc[:, :C] + bp[:, j * C:(j + 1) * C]
        q = acc[:, C:] + bq[:, j * C:(j + 1) * C]
        outs.append(jnp.tanh(p) * jax.nn.sigmoid(q))
    return outs


def _row_of_col0(m_ref, sel):
    return jax.lax.dot_general(m_ref[:, 0:1], sel, (((0,), (1,)), ((), ())),
                               preferred_element_type=F32)


def _fcmy(a_cat, t_rows, tos, g_cat, mask, bf):
    tot = sum(tos)
    ut = _eqf(_iota((t_rows * BN, t_rows), 0) // BN,
              _iota((t_rows * BN, t_rows), 1))
    uto = _eqf(_iota((tot * BN, tot), 0) // BN, _iota((tot * BN, tot), 1))
    big = _dot_bt(_dot(ut, a_cat), uto) * mask[:t_rows * BN, :]
    return _dot(big, g_cat) + bf


def _network(Rf, A, o_ref):
    wpq = jnp.concatenate([Rf("gtu_wp"), Rf("gtu_wq")], axis=1)
    bp = Rf("gtu_bp")
    bq = Rf("gtu_bq")
    cheb = Rf("cheb")
    theta = Rf("theta")

    x16 = Rf("x16")
    v96 = _dot_tb(x16, _eqf(_iota((B * CIN, B * CIN), 0),
                            _iota((B * CIN, B * CIN), 1)))
    sh = (T0 * BN, N * SEQ)
    perm = (_eqf(_iota(sh, 0) % N, _iota(sh, 1) // SEQ)
            * _eqf(_iota(sh, 0) // BN, _iota(sh, 1) % SEQ + (T0 - SEQ)))
    pv = _dot(perm, v96)
    mb = (T0 * BN, B * CIN)
    pv = pv * _eqf((_iota(mb, 0) // N) % B, _iota(mb, 1) // CIN)
    wb = _dot(_eqf(_iota((B * CIN, CIN), 0) % CIN, _iota((B * CIN, CIN), 1)),
              Rf("start_w"))
    x = _dot(pv, wb) + Rf("start_b")

    skip_last = jnp.zeros((BN, SC), F32) + Rf("skip0_b")
    rmat = _eqf(_iota((N * SEQ, SEQ), 0) % SEQ, _iota((N * SEQ, SEQ), 1))
    cm = (BN, N * SEQ)
    colmask = _eqf(_iota(cm, 1) // SEQ, _iota(cm, 0) % N)
    for cin in range(CIN):
        pick = _eqf(_iota((BN, B * CIN), 1),
                    (_iota((BN, B * CIN), 0) // N) * CIN + cin)
        g16 = _dot(pick, x16) * colmask
        a_c = _dot(g16, rmat)
        wsel = _eqf(_iota((SEQ, T0 * CIN), 0) * CIN + (T0 - SEQ) * CIN + cin,
                    _iota((SEQ, T0 * CIN), 1))
        skip_last = skip_last + _dot(a_c, _dot(wsel, Rf("skip0_w")))

    T = T0
    for i in range(NLAYERS):
        T_out = T - 6
        kw = T_out
        to_s = T - kw + 1
        tos = [T - k + 1 for k in GTU_KS]
        residual = x

        tot = sum(tos)
        km = (T * BN, tot * BN)
        kmask = _eqf(_iota(km, 0) % BN, _iota(km, 1) % BN)

        g = _gtu_bank(x, T, wpq, bp, bq)
        a1 = jnp.concatenate([A["l%d_A1_%d" % (i, j)] for j in range(3)],
                             axis=1)
        tc = _fcmy(a1, T, tos, jnp.concatenate(g, axis=0), kmask,
                   Rf("l%d_bF1" % i))
        x_new = jnp.maximum(x + tc, 0.0)

        s = jnp.zeros((BN, SC), F32) + Rf("l%d_skip_b" % i)
        skw = Rf("l%d_skip_w" % i)
        for dt in range(kw):
            r0 = (to_s - 1 + dt) * BN
            s = s + _dot(x_new[r0:r0 + BN, :], skw[dt * C:(dt + 1) * C, :])
        skip_last = skip_last + s

        r8t = Rf("l%d_R8T" % i)
        rnl = Rf("l%d_RNl" % i)
        mask_tt = Rf("l%d_MaskTT" % i)
        krep = Rf("l%d_Krep" % i)
        u2 = Rf("l%d_U2" % i)
        ve = Rf("l%d_Ve" % i)
        be = Rf("l%d_be" % i)
        w1row = Rf("l%d_W1row" % i)
        w2 = Rf("l%d_W2" % i)
        vs = Rf("l%d_Vs" % i)
        bs = Rf("l%d_bs" % i)
        selm = _eqf(_iota((T * N, N * C), 1),
                    (_iota((T * N, N * C), 0) % N) * C)
        u1rep = _row_of_col0(Rf("l%d_U1k" % i, ref=True), selm)
        selc = _eqf(_iota((C, N * C), 1), _iota((C, N * C), 0))
        u3row = _row_of_col0(Rf("l%d_U3k" % i, ref=True), selc)
        w3row = _row_of_col0(Rf("l%d_W3k" % i, ref=True), selc)

        xg_all = jnp.zeros((T * BN, C), F32)
        gsh = (T * N, T * BN)
        gr = _iota(gsh, 0)
        gq = _iota(gsh, 1)
        for b in range(B):
            gb = _eqf(gq, (gr // N) * BN + b * N + gr % N)
            xs = _dot(gb, x_new)
            lhs1 = _dot(_dot(r8t * u1rep, xs), u2)
            v3 = _dot_bt(u3row, xs)
            rhs1t = _dot(r8t * v3, rnl)
            prod1 = _dot_bt(lhs1, rhs1t)
            t_att = _softmax0(_dot(ve, jax.nn.sigmoid(prod1 + be)))
            w1t = _dot(w1row, t_att)
            k1 = krep * _dot(w1t, r8t)
            r = _dot(k1, xs)
            lhs2 = _dot(r, w2)
            vw3 = _dot_bt(w3row, xs)
            rhs2 = _dot(t_att, _dot(r8t * vw3, rnl))
            prod2 = _dot(lhs2, rhs2)
            s_att = _softmax0(_dot(vs, jax.nn.sigmoid(prod2 + bs)))
            acc = jnp.zeros((T * N, C), F32)
            for kk in range(K):
                a = cheb[kk * N:(kk + 1) * N, :] * s_att
                big = _dot_bt(_dot_bt(rnl, a), rnl) * mask_tt
                acc = acc + _dot(_dot(big, xs),
                                 theta[kk * C:(kk + 1) * C, :])
            xg_all = xg_all + _dot_tb(gb, jnp.maximum(acc, 0.0))

        g = _gtu_bank(xg_all, T, wpq, bp, bq)
        a2 = jnp.concatenate([A["l%d_A2_%d" % (i, j)] for j in range(3)],
                             axis=1)
        tc2 = _fcmy(a2, T_out, tos, jnp.concatenate(g, axis=0), kmask,
                    Rf("l%d_bF2" % i))
        off = (T - T_out) * BN
        xn2 = jnp.maximum(xg_all[off:, :] + tc2, 0.0) + residual[off:, :]
        rows = T_out * BN
        cnt = float(T_out * N * C)
        selb = _eqf((_iota((B, rows), 1) % BN) // N, _iota((B, rows), 0))
        ones_c = jnp.zeros((C, 1), F32) + 1.0
        mu8 = _dot(selb, _dot(xn2, ones_c)) / cnt
        d = xn2 - _dot_tb(selb, mu8)
        var8 = _dot(selb, _dot(d * d, ones_c)) / cnt
        x = d * jax.lax.rsqrt(_dot_tb(selb, var8) + EPS)
        T = T_out

    sk = _dot(x, Rf("skipE_w")) + Rf("skipE_b") + skip_last
    h = jnp.maximum(sk, 0.0)
    h = jnp.maximum(_dot(h, Rf("end1_w")) + Rf("end1_b"), 0.0)
    res = _dot(h, Rf("end2_w")) + Rf("end2_b")
    eye = _eqf(_iota((N, N), 0), _iota((N, N), 1))
    for b in range(B):
        o_ref[b] = _dot_tb(res[b * N:(b + 1) * N, :], eye)


def _body(idx, nin, *refs):
    nmat = len(_MROWS)
    o_ref = refs[nin]
    scratch = refs[nin + 1:]
    t = pl.program_id(0)

    def Rf(name, ref=False):
        r = refs[idx[name]]
        return r if ref else r[...]

    for m, (tr, _) in enumerate(_MROWS):
        slot = jnp.minimum(t, tr - 1) * 8
        scratch[m][pl.ds(slot, 8), :] = refs[m][...]

    @pl.when(t == T0)
    def _run():
        A = {}
        pos = 0
        for i in range(NLAYERS):
            for half, tag in ((0, "A1"), (1, "A2")):
                for j, k in enumerate(GTU_KS):
                    m = pos + half * 3 + j
                    tr, to = _MROWS[m]
                    s = scratch[m][...]
                    rowsel = _eqf(_iota((tr, tr * 8), 1),
                                  _iota((tr, tr * 8), 0) * 8)
                    colsel = _eqf(_iota((to * BN, to), 0),
                                  _iota((to * BN, to), 1) * BN)
                    A["l%d_%s_%d" % (i, tag, j)] = _dot(_dot(rowsel, s),
                                                        colsel)
            pos += 6
        _network(Rf, A, o_ref)


def kernel(x_in, start_w, skip0_w, gtu_wp, gtu_wq, theta, cheb, start_b, skip0_b, gtu_bp, gtu_bq, l0_skip_w, l0_skip_b, l0_M1_0, l0_M1_1, l0_M1_2, l0_M2_0, l0_M2_1, l0_M2_2, l0_bF1, l0_bF2, l0_U1k, l0_U2, l0_U3k, l0_Ve, l0_be, l0_W1row, l0_W2, l0_W3k, l0_Vs, l0_bs, l0_R8T, l0_Krep, l0_RNl, l0_MaskTT, l1_skip_w, l1_skip_b, l1_M1_0, l1_M1_1, l1_M1_2, l1_M2_0, l1_M2_1, l1_M2_2, l1_bF1, l1_bF2, l1_U1k, l1_U2, l1_U3k, l1_Ve, l1_be, l1_W1row, l1_W2, l1_W3k, l1_Vs, l1_bs, l1_R8T, l1_Krep, l1_RNl, l1_MaskTT, l2_skip_w, l2_skip_b, l2_M1_0, l2_M1_1, l2_M1_2, l2_M2_0, l2_M2_1, l2_M2_2, l2_bF1, l2_bF2, l2_U1k, l2_U2, l2_U3k, l2_Ve, l2_be, l2_W1row, l2_W2, l2_W3k, l2_Vs, l2_bs, l2_R8T, l2_Krep, l2_RNl, l2_MaskTT, skipE_w, end1_w, end2_w, skipE_b, end1_b, end2_b):
    lraw = [
        dict(skip_w=l0_skip_w, skip_b=l0_skip_b,
             M1=(l0_M1_0, l0_M1_1, l0_M1_2), M2=(l0_M2_0, l0_M2_1, l0_M2_2),
             bF1=l0_bF1, bF2=l0_bF2, U1k=l0_U1k, U2=l0_U2, U3k=l0_U3k,
             Ve=l0_Ve, be=l0_be, W1row=l0_W1row, W2=l0_W2, W3k=l0_W3k,
             Vs=l0_Vs, bs=l0_bs, R8T=l0_R8T, Krep=l0_Krep, RNl=l0_RNl,
             MaskTT=l0_MaskTT),
        dict(skip_w=l1_skip_w, skip_b=l1_skip_b,
             M1=(l1_M1_0, l1_M1_1, l1_M1_2), M2=(l1_M2_0, l1_M2_1, l1_M2_2),
             bF1=l1_bF1, bF2=l1_bF2, U1k=l1_U1k, U2=l1_U2, U3k=l1_U3k,
             Ve=l1_Ve, be=l1_be, W1row=l1_W1row, W2=l1_W2, W3k=l1_W3k,
             Vs=l1_Vs, bs=l1_bs, R8T=l1_R8T, Krep=l1_Krep, RNl=l1_RNl,
             MaskTT=l1_MaskTT),
        dict(skip_w=l2_skip_w, skip_b=l2_skip_b,
             M1=(l2_M1_0, l2_M1_1, l2_M1_2), M2=(l2_M2_0, l2_M2_1, l2_M2_2),
             bF1=l2_bF1, bF2=l2_bF2, U1k=l2_U1k, U2=l2_U2, U3k=l2_U3k,
             Ve=l2_Ve, be=l2_be, W1row=l2_W1row, W2=l2_W2, W3k=l2_W3k,
             Vs=l2_Vs, bs=l2_bs, R8T=l2_R8T, Krep=l2_Krep, RNl=l2_RNl,
             MaskTT=l2_MaskTT),
    ]

    args = []
    specs = []
    idx = {}
    scratch_shapes = []

    for lp in lraw:
        for mat in list(lp["M1"]) + list(lp["M2"]):
            args.append(mat)
            specs.append(pl.BlockSpec(
                (8, mat.shape[1]),
                lambda t, _r=mat.shape[0] // BN: (jnp.minimum(t, _r - 1) * 8,
                                                  0)))
            scratch_shapes.append(
                pltpu.VMEM((mat.shape[0] // BN * 8, mat.shape[1]), F32))

    def add(name, arr):
        idx[name] = len(args)
        args.append(arr)
        nd = arr.ndim
        specs.append(pl.BlockSpec(arr.shape, lambda t, _n=nd: (0,) * _n))

    add("x16", x_in.reshape(B * CIN, N * SEQ))
    for nm, arr in (("start_w", start_w), ("start_b", start_b),
                    ("skip0_w", skip0_w), ("skip0_b", skip0_b),
                    ("gtu_wp", gtu_wp), ("gtu_wq", gtu_wq),
                    ("gtu_bp", gtu_bp), ("gtu_bq", gtu_bq),
                    ("cheb", cheb), ("theta", theta),
                    ("skipE_w", skipE_w), ("skipE_b", skipE_b),
                    ("end1_w", end1_w), ("end1_b", end1_b),
                    ("end2_w", end2_w), ("end2_b", end2_b)):
        add(nm, arr)
    for i, lp in enumerate(lraw):
        for nm in ("bF1", "bF2", "skip_w", "skip_b", "U1k", "U2", "U3k",
                   "Ve", "be", "W1row", "W2", "W3k", "Vs", "bs", "R8T",
                   "Krep", "RNl", "MaskTT"):
            add("l%d_%s" % (i, nm), lp[nm])

    nin = len(args)
    out = pl.pallas_call(
        functools.partial(_body, idx, nin),
        out_shape=jax.ShapeDtypeStruct((B, OUT_DIM, N), F32),
        grid=(T0 + 1,),
        in_specs=specs,
        out_specs=pl.BlockSpec((B, OUT_DIM, N), lambda t: (0, 0, 0)),
        scratch_shapes=scratch_shapes,
        compiler_params=pltpu.CompilerParams(
            dimension_semantics=("arbitrary",)),
    )(*args)
    return out[..., None]


# speedup vs baseline: 1.0908x; 1.0908x over previous
import functools

import jax
import jax.numpy as jnp
from jax.experimental import pallas as pl
from jax.experimental.pallas import tpu as pltpu

F32 = jnp.float32
GTU_KS = (3, 5, 7)

B = 8
N = 8
BN = B * N
CIN = 2
C = 32
SC = 64
SEQ = 12
T0 = 19
NLAYERS = 3
K = 3
OUT_DIM = 12
EPS = 1e-5

TS = (19, 13, 7)

_MROWS = []
for _i in range(NLAYERS):
    _t = TS[_i]
    for _k in GTU_KS:
        _MROWS.append((_t, _t - _k + 1))
    for _k in GTU_KS:
        _MROWS.append((_t - 6, _t - _k + 1))


def _dot(a, b):
    return jnp.dot(a, b, preferred_element_type=F32)


def _dot_bt(a, b):
    return jax.lax.dot_general(a, b, (((1,), (1,)), ((), ())),
                               preferred_element_type=F32)


def _dot_tb(a, b):
    return jax.lax.dot_general(a, b, (((0,), (0,)), ((), ())),
                               preferred_element_type=F32)


def _softmax0(x):
    m = jnp.max(x, axis=0, keepdims=True)
    e = jnp.exp(x - m)
    return e / jnp.sum(e, axis=0, keepdims=True)


def _iota(shape, d):
    return jax.lax.broadcasted_iota(jnp.int32, shape, d)


def _eqf(a, b):
    return (a == b).astype(F32)


def _gtu_bank(X, T, wpq, bp, bq):
    outs = []
    tap = 0
    for j, k in enumerate(GTU_KS):
        rows = (T - k + 1) * BN
        acc = jnp.zeros((rows, 2 * C), F32)
        for dt in range(k):
            acc = acc + _dot(X[dt * BN: dt * BN + rows, :],
                             wpq[(tap + dt) * C:(tap + dt + 1) * C, :])
        tap += k
        p = acc[:, :C] + bp[:, j * C:(j + 1) * C]
        q = acc[:, C:] + bq[:, j * C:(j + 1) * C]
        outs.append(jnp.tanh(p) * jax.nn.sigmoid(q))
    return outs


def _row_of_col0(m_ref, sel):
    return jax.lax.dot_general(m_ref[:, 0:1], sel, (((0,), (1,)), ((), ())),
                               preferred_element_type=F32)


def _fcmy(a_cat, t_rows, tos, g_cat, mask, bf):
    tot = sum(tos)
    ut = _eqf(_iota((t_rows * BN, t_rows), 0) // BN,
              _iota((t_rows * BN, t_rows), 1))
    uto = _eqf(_iota((tot * BN, tot), 0) // BN, _iota((tot * BN, tot), 1))
    big = _dot_bt(_dot(ut, a_cat), uto) * mask[:t_rows * BN, :]
    return _dot(big, g_cat) + bf


def _network(Rf, A, o_ref):
    wpq = jnp.concatenate([Rf("gtu_wp"), Rf("gtu_wq")], axis=1)
    bp = Rf("gtu_bp")
    bq = Rf("gtu_bq")
    cheb = Rf("cheb")
    theta = Rf("theta")

    x16 = Rf("x16")
    v96 = _dot_tb(x16, _eqf(_iota((B * CIN, B * CIN), 0),
                            _iota((B * CIN, B * CIN), 1)))
    sh = (T0 * BN, N * SEQ)
    perm = (_eqf(_iota(sh, 0) % N, _iota(sh, 1) // SEQ)
            * _eqf(_iota(sh, 0) // BN, _iota(sh, 1) % SEQ + (T0 - SEQ)))
    pv = _dot(perm, v96)
    mb = (T0 * BN, B * CIN)
    pv = pv * _eqf((_iota(mb, 0) // N) % B, _iota(mb, 1) // CIN)
    wb = _dot(_eqf(_iota((B * CIN, CIN), 0) % CIN, _iota((B * CIN, CIN), 1)),
              Rf("start_w"))
    x = _dot(pv, wb) + Rf("start_b")

    skip_last = jnp.zeros((BN, SC), F32) + Rf("skip0_b")
    rmat = _eqf(_iota((N * SEQ, SEQ), 0) % SEQ, _iota((N * SEQ, SEQ), 1))
    cm = (BN, N * SEQ)
    colmask = _eqf(_iota(cm, 1) // SEQ, _iota(cm, 0) % N)
    for cin in range(CIN):
        pick = _eqf(_iota((BN, B * CIN), 1),
                    (_iota((BN, B * CIN), 0) // N) * CIN + cin)
        g16 = _dot(pick, x16) * colmask
        a_c = _dot(g16, rmat)
        wsel = _eqf(_iota((SEQ, T0 * CIN), 0) * CIN + (T0 - SEQ) * CIN + cin,
                    _iota((SEQ, T0 * CIN), 1))
        skip_last = skip_last + _dot(a_c, _dot(wsel, Rf("skip0_w")))

    T = T0
    for i in range(NLAYERS):
        T_out = T - 6
        kw = T_out
        to_s = T - kw + 1
        tos = [T - k + 1 for k in GTU_KS]
        residual = x

        tot = sum(tos)
        km = (T * BN, tot * BN)
        kmask = _eqf(_iota(km, 0) % BN, _iota(km, 1) % BN)

        g = _gtu_bank(x, T, wpq, bp, bq)
        a1 = jnp.concatenate([A["l%d_A1_%d" % (i, j)] for j in range(3)],
                             axis=1)
        tc = _fcmy(a1, T, tos, jnp.concatenate(g, axis=0), kmask,
                   Rf("l%d_bF1" % i))
        x_new = jnp.maximum(x + tc, 0.0)

        s = jnp.zeros((BN, SC), F32) + Rf("l%d_skip_b" % i)
        skw = Rf("l%d_skip_w" % i)
        for dt in range(kw):
            r0 = (to_s - 1 + dt) * BN
            s = s + _dot(x_new[r0:r0 + BN, :], skw[dt * C:(dt + 1) * C, :])
        skip_last = skip_last + s

        r8t = Rf("l%d_R8T" % i)
        rnl = Rf("l%d_RNl" % i)
        mask_tt = Rf("l%d_MaskTT" % i)
        krep = Rf("l%d_Krep" % i)
        u2 = Rf("l%d_U2" % i)
        ve = Rf("l%d_Ve" % i)
        be = Rf("l%d_be" % i)
        w1row = Rf("l%d_W1row" % i)
        w2 = Rf("l%d_W2" % i)
        vs = Rf("l%d_Vs" % i)
        bs = Rf("l%d_bs" % i)
        selm = _eqf(_iota((T * N, N * C), 1),
                    (_iota((T * N, N * C), 0) % N) * C)
        u1rep = _row_of_col0(Rf("l%d_U1k" % i, ref=True), selm)
        selc = _eqf(_iota((C, N * C), 1), _iota((C, N * C), 0))
        u3row = _row_of_col0(Rf("l%d_U3k" % i, ref=True), selc)
        w3row = _row_of_col0(Rf("l%d_W3k" % i, ref=True), selc)

        xg_all = jnp.zeros((T * BN, C), F32)
        gsh = (T * N, T * BN)
        gr = _iota(gsh, 0)
        gq = _iota(gsh, 1)
        for b in range(B):
            gb = _eqf(gq, (gr // N) * BN + b * N + gr % N)
            xs = _dot(gb, x_new)
            lhs1 = _dot(_dot(r8t * u1rep, xs), u2)
            v3 = _dot_bt(u3row, xs)
            rhs1t = _dot(r8t * v3, rnl)
            prod1 = _dot_bt(lhs1, rhs1t)
            t_att = _softmax0(_dot(ve, jax.nn.sigmoid(prod1 + be)))
            w1t = _dot(w1row, t_att)
            k1 = krep * _dot(w1t, r8t)
            r = _dot(k1, xs)
            lhs2 = _dot(r, w2)
            vw3 = _dot_bt(w3row, xs)
            rhs2 = _dot(t_att, _dot(r8t * vw3, rnl))
            prod2 = _dot(lhs2, rhs2)
            s_att = _softmax0(_dot(vs, jax.nn.sigmoid(prod2 + bs)))
            acc = jnp.zeros((T * N, C), F32)
            for kk in range(K):
                a = cheb[kk * N:(kk + 1) * N, :] * s_att
                big = _dot_bt(_dot_bt(rnl, a), rnl) * mask_tt
                acc = acc + _dot(_dot(big, xs),
                                 theta[kk * C:(kk + 1) * C, :])
            xg_all = xg_all + _dot_tb(gb, jnp.maximum(acc, 0.0))

        g = _gtu_bank(xg_all, T, wpq, bp, bq)
        a2 = jnp.concatenate([A["l%d_A2_%d" % (i, j)] for j in range(3)],
                             axis=1)
        tc2 = _fcmy(a2, T_out, tos, jnp.concatenate(g, axis=0), kmask,
                    Rf("l%d_bF2" % i))
        off = (T - T_out) * BN
        xn2 = jnp.maximum(xg_all[off:, :] + tc2, 0.0) + residual[off:, :]
        rows = T_out * BN
        cnt = float(T_out * N * C)
        selb = _eqf((_iota((B, rows), 1) % BN) // N, _iota((B, rows), 0))
        ones_c = jnp.zeros((C, 1), F32) + 1.0
        mu8 = _dot(selb, _dot(xn2, ones_c)) / cnt
        d = xn2 - _dot_tb(selb, mu8)
        var8 = _dot(selb, _dot(d * d, ones_c)) / cnt
        x = d * jax.lax.rsqrt(_dot_tb(selb, var8) + EPS)
        T = T_out

    sk = _dot(x, Rf("skipE_w")) + Rf("skipE_b") + skip_last
    h = jnp.maximum(sk, 0.0)
    h = jnp.maximum(_dot(h, Rf("end1_w")) + Rf("end1_b"), 0.0)
    res = _dot(h, Rf("end2_w")) + Rf("end2_b")
    eye = _eqf(_iota((N, N), 0), _iota((N, N), 1))
    for b in range(B):
        o_ref[b] = _dot_tb(res[b * N:(b + 1) * N, :], eye)


def _body(idx, nin, *refs):
    nmat = len(_MROWS)
    o_ref = refs[nin]
    scratch = refs[nin + 1:nin + 1 + nmat]
    sem = refs[nin + 1 + nmat]

    def Rf(name, ref=False):
        r = refs[idx[name]]
        return r if ref else r[...]

    for m, (tr, _) in enumerate(_MROWS):
        copies = []
        for t in range(tr):
            c = pltpu.make_async_copy(
                refs[m].at[pl.ds(t * BN, 8), :],
                scratch[m].at[pl.ds(t * 8, 8), :], sem)
            c.start()
            copies.append(c)
        for c in copies:
            c.wait()

    A = {}
    pos = 0
    for i in range(NLAYERS):
        for half, tag in ((0, "A1"), (1, "A2")):
            for j, k in enumerate(GTU_KS):
                m = pos + half * 3 + j
                tr, to = _MROWS[m]
                s = scratch[m][...]
                rowsel = _eqf(_iota((tr, tr * 8), 1),
                              _iota((tr, tr * 8), 0) * 8)
                colsel = _eqf(_iota((to * BN, to), 0),
                              _iota((to * BN, to), 1) * BN)
                A["l%d_%s_%d" % (i, tag, j)] = _dot(_dot(rowsel, s), colsel)
        pos += 6
    _network(Rf, A, o_ref)


def kernel(x_in, start_w, skip0_w, gtu_wp, gtu_wq, theta, cheb, start_b, skip0_b, gtu_bp, gtu_bq, l0_skip_w, l0_skip_b, l0_M1_0, l0_M1_1, l0_M1_2, l0_M2_0, l0_M2_1, l0_M2_2, l0_bF1, l0_bF2, l0_U1k, l0_U2, l0_U3k, l0_Ve, l0_be, l0_W1row, l0_W2, l0_W3k, l0_Vs, l0_bs, l0_R8T, l0_Krep, l0_RNl, l0_MaskTT, l1_skip_w, l1_skip_b, l1_M1_0, l1_M1_1, l1_M1_2, l1_M2_0, l1_M2_1, l1_M2_2, l1_bF1, l1_bF2, l1_U1k, l1_U2, l1_U3k, l1_Ve, l1_be, l1_W1row, l1_W2, l1_W3k, l1_Vs, l1_bs, l1_R8T, l1_Krep, l1_RNl, l1_MaskTT, l2_skip_w, l2_skip_b, l2_M1_0, l2_M1_1, l2_M1_2, l2_M2_0, l2_M2_1, l2_M2_2, l2_bF1, l2_bF2, l2_U1k, l2_U2, l2_U3k, l2_Ve, l2_be, l2_W1row, l2_W2, l2_W3k, l2_Vs, l2_bs, l2_R8T, l2_Krep, l2_RNl, l2_MaskTT, skipE_w, end1_w, end2_w, skipE_b, end1_b, end2_b):
    lraw = [
        dict(skip_w=l0_skip_w, skip_b=l0_skip_b,
             M1=(l0_M1_0, l0_M1_1, l0_M1_2), M2=(l0_M2_0, l0_M2_1, l0_M2_2),
             bF1=l0_bF1, bF2=l0_bF2, U1k=l0_U1k, U2=l0_U2, U3k=l0_U3k,
             Ve=l0_Ve, be=l0_be, W1row=l0_W1row, W2=l0_W2, W3k=l0_W3k,
             Vs=l0_Vs, bs=l0_bs, R8T=l0_R8T, Krep=l0_Krep, RNl=l0_RNl,
             MaskTT=l0_MaskTT),
        dict(skip_w=l1_skip_w, skip_b=l1_skip_b,
             M1=(l1_M1_0, l1_M1_1, l1_M1_2), M2=(l1_M2_0, l1_M2_1, l1_M2_2),
             bF1=l1_bF1, bF2=l1_bF2, U1k=l1_U1k, U2=l1_U2, U3k=l1_U3k,
             Ve=l1_Ve, be=l1_be, W1row=l1_W1row, W2=l1_W2, W3k=l1_W3k,
             Vs=l1_Vs, bs=l1_bs, R8T=l1_R8T, Krep=l1_Krep, RNl=l1_RNl,
             MaskTT=l1_MaskTT),
        dict(skip_w=l2_skip_w, skip_b=l2_skip_b,
             M1=(l2_M1_0, l2_M1_1, l2_M1_2), M2=(l2_M2_0, l2_M2_1, l2_M2_2),
             bF1=l2_bF1, bF2=l2_bF2, U1k=l2_U1k, U2=l2_U2, U3k=l2_U3k,
             Ve=l2_Ve, be=l2_be, W1row=l2_W1row, W2=l2_W2, W3k=l2_W3k,
             Vs=l2_Vs, bs=l2_bs, R8T=l2_R8T, Krep=l2_Krep, RNl=l2_RNl,
             MaskTT=l2_MaskTT),
    ]

    args = []
    specs = []
    idx = {}
    scratch_shapes = []

    for lp in lraw:
        for mat in list(lp["M1"]) + list(lp["M2"]):
            args.append(mat)
            specs.append(pl.BlockSpec(memory_space=pltpu.MemorySpace.HBM))
            scratch_shapes.append(
                pltpu.VMEM((mat.shape[0] // BN * 8, mat.shape[1]), F32))

    def add(name, arr):
        idx[name] = len(args)
        args.append(arr)
        nd = arr.ndim
        specs.append(pl.BlockSpec(arr.shape, lambda t, _n=nd: (0,) * _n))

    add("x16", x_in.reshape(B * CIN, N * SEQ))
    for nm, arr in (("start_w", start_w), ("start_b", start_b),
                    ("skip0_w", skip0_w), ("skip0_b", skip0_b),
                    ("gtu_wp", gtu_wp), ("gtu_wq", gtu_wq),
                    ("gtu_bp", gtu_bp), ("gtu_bq", gtu_bq),
                    ("cheb", cheb), ("theta", theta),
                    ("skipE_w", skipE_w), ("skipE_b", skipE_b),
                    ("end1_w", end1_w), ("end1_b", end1_b),
                    ("end2_w", end2_w), ("end2_b", end2_b)):
        add(nm, arr)
    for i, lp in enumerate(lraw):
        for nm in ("bF1", "bF2", "skip_w", "skip_b", "U1k", "U2", "U3k",
                   "Ve", "be", "W1row", "W2", "W3k", "Vs", "bs", "R8T",
                   "Krep", "RNl", "MaskTT"):
            add("l%d_%s" % (i, nm), lp[nm])

    scratch_shapes.append(pltpu.SemaphoreType.DMA)
    nin = len(args)
    out = pl.pallas_call(
        functools.partial(_body, idx, nin),
        out_shape=jax.ShapeDtypeStruct((B, OUT_DIM, N), F32),
        grid=(1,),
        in_specs=specs,
        out_specs=pl.BlockSpec((B, OUT_DIM, N), lambda t: (0, 0, 0)),
        scratch_shapes=scratch_shapes,
        compiler_params=pltpu.CompilerParams(
            dimension_semantics=("arbitrary",)),
    )(*args)
    return out[..., None]
